# baseline (speedup 1.0000x reference)
import jax
import jax.numpy as jnp
from jax.experimental import pallas as pl

_BR = 128


def _embed_body(ts_ref, nv_ref, mk_ref, wd_ref, wv_ref, bd_ref, bv_ref,
                tb_ref, out_ref):
    D = wd_ref.shape[0]
    L = ts_ref.shape[1]
    bias = bd_ref[...] + bv_ref[...] + tb_ref[...]
    wd = wd_ref[...]
    wv = wv_ref[...]
    for c in range(_BR):
        row = slice(c, c + 1)
        ts = jnp.broadcast_to(ts_ref[row, :], (D, L))
        nv = jnp.broadcast_to(nv_ref[row, :], (D, L))
        mk = jnp.broadcast_to(mk_ref[row, :], (D, L))
        out_ref[pl.ds(c * D, D)] = (ts * wd + nv * wv + bias) * mk


def kernel(timestamp, numerical_value, mask, code, W_date, b_date, table,
           W_val, b_val):
    B, L = timestamp.shape
    D = W_date.shape[0]
    del code

    row_spec = pl.BlockSpec((_BR, L), lambda i: (i, 0))
    col_spec = pl.BlockSpec((D, 1), lambda i: (0, 0))

    out2 = pl.pallas_call(
        _embed_body,
        grid=(B // _BR,),
        in_specs=[row_spec, row_spec, row_spec,
                  col_spec, col_spec, col_spec, col_spec, col_spec],
        out_specs=pl.BlockSpec((_BR * D, L), lambda i: (i, 0)),
        out_shape=jax.ShapeDtypeStruct((B * D, L), jnp.float32),
    )(timestamp, numerical_value, mask.reshape(B, L),
      W_date, W_val,
      b_date.reshape(D, 1), b_val.reshape(D, 1), table.reshape(D, 1))
    return out2.reshape(B, D, L)
